# baseline (speedup 1.0000x reference)
import jax
import jax.numpy as jnp
from jax.experimental import pallas as pl
from jax.experimental.pallas import tpu as pltpu

HIDDEN = 128
IN_EXT = 5
LANE = 128
CHUNK = 4096


def _cdiv(a, b):
    return (a + b - 1) // b


def _fused_kernel(xa_ref, w1e_ref, w2t_ref, b2_ref, o_ref):
    w1e = w1e_ref[...]
    w2t = w2t_ref[...]
    b2 = b2_ref[0, 0]
    tb = o_ref.shape[-1]
    for c in range(tb // CHUNK):
        sl = pl.ds(c * CHUNK, CHUNK)
        h = jnp.dot(w1e, xa_ref[:, sl],
                    preferred_element_type=jnp.float32)
        hb = jnp.maximum(h.astype(jnp.bfloat16), jnp.bfloat16(0.0))
        q = jnp.dot(w2t, hb,
                    preferred_element_type=jnp.float32)
        o_ref[:, sl] = q + b2


def kernel(x, a, w1, b1, w2, b2):
    B = x.shape[0]
    TB = 65536
    nt = _cdiv(B, TB)
    if nt > 1 and nt % 2 == 1:
        nt += 1
    B_pad = nt * TB

    ones = jnp.ones((B, 1), x.dtype)
    xa = jnp.concatenate([x, a, ones], axis=-1)
    if B_pad != B:
        xa = jnp.pad(xa, ((0, B_pad - B), (0, 0)))
    xa_t = xa.T.astype(jnp.bfloat16)

    w1e = jnp.concatenate([w1, b1.reshape(1, HIDDEN)],
                          axis=0).T.astype(jnp.bfloat16)
    w2t = w2.reshape(1, HIDDEN).astype(jnp.bfloat16)
    b2s = b2.reshape(1, 1)

    q_t = pl.pallas_call(
        _fused_kernel,
        out_shape=jax.ShapeDtypeStruct((1, B_pad), jnp.float32),
        grid=(nt,),
        in_specs=[
            pl.BlockSpec((IN_EXT, TB), lambda i: (0, i)),
            pl.BlockSpec((HIDDEN, IN_EXT), lambda i: (0, 0)),
            pl.BlockSpec((1, HIDDEN), lambda i: (0, 0)),
            pl.BlockSpec((1, 1), lambda i: (0, 0),
                         memory_space=pltpu.SMEM),
        ],
        out_specs=pl.BlockSpec((1, TB), lambda i: (0, i)),
        compiler_params=pltpu.CompilerParams(
            dimension_semantics=("parallel",)),
    )(xa_t, w1e, w2t, b2s)

    return q_t.reshape(B_pad, 1)[:B]


# speedup vs baseline: 1.1643x; 1.1643x over previous
import jax
import jax.numpy as jnp
from jax.experimental import pallas as pl
from jax.experimental.pallas import tpu as pltpu

HIDDEN = 128
IN_EXT = 5
LANE = 128


def _cdiv(a, b):
    return (a + b - 1) // b


def _fused_kernel(xt_ref, af_ref, w1e_ref, w2t_ref, b2_ref, o_ref):
    a16 = af_ref[...].astype(jnp.bfloat16)
    ones = jnp.ones(a16.shape, jnp.bfloat16)
    xa = jnp.concatenate([xt_ref[...], a16, ones], axis=0)
    h = jnp.dot(w1e_ref[...], xa,
                preferred_element_type=jnp.float32)
    h = jnp.maximum(h.astype(jnp.bfloat16), jnp.bfloat16(0.0))
    q = jnp.dot(w2t_ref[...], h,
                preferred_element_type=jnp.float32)
    o_ref[...] = q + b2_ref[0, 0]


def kernel(x, a, w1, b1, w2, b2):
    B = x.shape[0]
    TB = 65536
    nt = _cdiv(B, TB)
    if nt > 1 and nt % 2 == 1:
        nt += 1
    B_pad = nt * TB

    xt = x.T.astype(jnp.bfloat16)
    af = a.reshape(1, B)
    if B_pad != B:
        xt = jnp.pad(xt, ((0, 0), (0, B_pad - B)))
        af = jnp.pad(af, ((0, 0), (0, B_pad - B)))

    w1e = jnp.concatenate([w1, b1.reshape(1, HIDDEN)],
                          axis=0).T.astype(jnp.bfloat16)
    w2t = w2.reshape(1, HIDDEN).astype(jnp.bfloat16)
    b2s = b2.reshape(1, 1)

    q_t = pl.pallas_call(
        _fused_kernel,
        out_shape=jax.ShapeDtypeStruct((1, B_pad), jnp.float32),
        grid=(nt,),
        in_specs=[
            pl.BlockSpec((3, TB), lambda i: (0, i)),
            pl.BlockSpec((1, TB), lambda i: (0, i)),
            pl.BlockSpec((HIDDEN, IN_EXT), lambda i: (0, 0)),
            pl.BlockSpec((1, HIDDEN), lambda i: (0, 0)),
            pl.BlockSpec((1, 1), lambda i: (0, 0),
                         memory_space=pltpu.SMEM),
        ],
        out_specs=pl.BlockSpec((1, TB), lambda i: (0, i)),
        compiler_params=pltpu.CompilerParams(
            dimension_semantics=("parallel",)),
    )(xt, af, w1e, w2t, b2s)

    return q_t.reshape(B_pad, 1)[:B]
